# baseline (speedup 1.0000x reference)
import functools

import jax
import jax.numpy as jnp
from jax import lax
from jax.experimental import pallas as pl
from jax.experimental.pallas import tpu as pltpu
from jax.experimental.pallas import tpu_sc as plsc

_PADW = 128
_BV = 2048
_BG = 128
_LANES = 16


def _relayout_block(t_ref, o_ref, *, scale):
    o_ref[:, 0:t_ref.shape[0]] = t_ref[...].T * scale


@functools.lru_cache(maxsize=None)
def _build_relayout(vocab, emb):
    grid = (vocab + _BV - 1) // _BV
    return pl.pallas_call(
        functools.partial(_relayout_block, scale=float(emb) ** 0.5),
        grid=(grid,),
        in_specs=[pl.BlockSpec((emb, _BV), lambda i: (0, i))],
        out_specs=pl.BlockSpec((_BV, _PADW), lambda i: (i, 0)),
        out_shape=jax.ShapeDtypeStruct((vocab, _PADW), jnp.float32),
    )


@functools.lru_cache(maxsize=None)
def _build_gather(vocab, emb, bsz, seq):
    info = plsc.get_sparse_core_info()
    nc, ns = info.num_cores, info.num_subcores
    nw = nc * ns
    assert bsz == nw * _BG
    assert seq % 2 == 0 and emb % 8 == 0
    ne = emb // 8
    mesh = plsc.VectorSubcoreMesh(core_axis_name="c", subcore_axis_name="s")

    @functools.partial(
        pl.kernel,
        out_type=jax.ShapeDtypeStruct((seq, ne, nw, 8, _BG), jnp.float32),
        mesh=mesh,
        compiler_params=pltpu.CompilerParams(use_tc_tiling_on_sc=False,
                                             needs_layout_passes=False),
        scratch_types=[
            pltpu.VMEM((seq, _BG), jnp.int32),
            pltpu.VMEM((_BG, _PADW), jnp.float32),
            pltpu.VMEM((_BG, _PADW), jnp.float32),
            pltpu.VMEM((emb, _BG), jnp.float32),
            pltpu.VMEM((emb, _BG), jnp.float32),
            pltpu.SemaphoreType.DMA,
            pltpu.SemaphoreType.DMA,
            pltpu.SemaphoreType.DMA,
            pltpu.SemaphoreType.DMA,
        ],
    )
    def emb_kernel(table_hbm, ids_hbm, out_hbm,
                   ids_v, rows0, rows1, ot0, ot1,
                   gsem0, gsem1, osem0, osem1):
        wid = lax.axis_index("s") * nc + lax.axis_index("c")
        bufs = ((rows0, ot0, gsem0, osem0), (rows1, ot1, gsem1, osem1))

        pltpu.sync_copy(ids_hbm.at[wid], ids_v)

        def gather(b, l):
            rows_v, _, gsem, _ = bufs[b]
            return pltpu.make_async_copy(table_hbm.at[ids_v.at[l]],
                                         rows_v, gsem)

        def out_copies(b, l):
            _, ot, _, osem = bufs[b]
            return [pltpu.make_async_copy(ot.at[pl.ds(eg * 8, 8)],
                                          out_hbm.at[l, eg, wid], osem)
                    for eg in range(ne)]

        ridx = [lax.iota(jnp.int32, _LANES) + i * _LANES
                for i in range(_BG // _LANES)]

        for b in range(2):
            gather(b, b).start()

        @pl.loop(0, seq, step=2)
        def _l_loop(g):
            for b in range(2):
                l = g + b
                rows_v, ot, _, _ = bufs[b]
                gather(b, l).wait()

                @pl.when(l >= 2)
                def _drain():
                    for cp in out_copies(b, l):
                        cp.wait()

                @pl.loop(0, emb, step=4)
                def _transpose(e0):
                    for de in range(4):
                        e = e0 + de
                        cidx = jnp.broadcast_to(e, (_LANES,)).astype(jnp.int32)
                        for i in range(_BG // _LANES):
                            ot[e, pl.ds(i * _LANES, _LANES)] = (
                                plsc.load_gather(rows_v, [ridx[i], cidx]))

                for cp in out_copies(b, l):
                    cp.start()

                @pl.when(l + 2 < seq)
                def _prefetch():
                    gather(b, l + 2).start()

        for b in range(2):
            for cp in out_copies(b, seq - 2 + b):
                cp.wait()

    return emb_kernel


def kernel(token_ids, table):
    bsz, seq = token_ids.shape
    vocab, emb = table.shape
    nw = bsz // _BG
    table_pad = _build_relayout(vocab, emb)(table.T)
    ids = jnp.transpose(token_ids.T.reshape(seq, nw, _BG), (1, 0, 2))
    ids = ids.astype(jnp.int32)
    out5 = _build_gather(vocab, emb, bsz, seq)(table_pad, ids)
    out = jnp.transpose(out5, (2, 4, 0, 1, 3)).reshape(bsz, seq, emb)
    return out


# speedup vs baseline: 1.9148x; 1.9148x over previous
import functools

import jax
import jax.numpy as jnp
from jax import lax
from jax.experimental import pallas as pl
from jax.experimental.pallas import tpu as pltpu
from jax.experimental.pallas import tpu_sc as plsc

_PADW = 128
_BV = 2048
_BG = 128
_LANES = 16


def _relayout_block(t_ref, o_ref, *, scale):
    o_ref[:, 0:t_ref.shape[0]] = t_ref[...].T * scale


@functools.lru_cache(maxsize=None)
def _build_relayout(vocab, emb):
    grid = (vocab + _BV - 1) // _BV
    return pl.pallas_call(
        functools.partial(_relayout_block, scale=float(emb) ** 0.5),
        grid=(grid,),
        in_specs=[pl.BlockSpec((emb, _BV), lambda i: (0, i))],
        out_specs=pl.BlockSpec((_BV, _PADW), lambda i: (i, 0)),
        out_shape=jax.ShapeDtypeStruct((vocab, _PADW), jnp.float32),
    )


@functools.lru_cache(maxsize=None)
def _build_gather(vocab, emb, bsz, seq):
    info = plsc.get_sparse_core_info()
    nc, ns = info.num_cores, info.num_subcores
    nw = nc * ns
    assert bsz == nw * _BG
    assert seq % 2 == 0 and emb % 8 == 0
    ne = emb // 8
    mesh = plsc.VectorSubcoreMesh(core_axis_name="c", subcore_axis_name="s")

    @functools.partial(
        pl.kernel,
        out_type=jax.ShapeDtypeStruct((seq, ne, nw, 8, _BG), jnp.float32),
        mesh=mesh,
        compiler_params=pltpu.CompilerParams(use_tc_tiling_on_sc=False,
                                             needs_layout_passes=False),
        scratch_types=[
            pltpu.VMEM((seq, _BG), jnp.int32),
            pltpu.VMEM((_BG, _PADW), jnp.float32),
            pltpu.VMEM((_BG, _PADW), jnp.float32),
            pltpu.VMEM((emb, _BG), jnp.float32),
            pltpu.VMEM((emb, _BG), jnp.float32),
            pltpu.SemaphoreType.DMA,
            pltpu.SemaphoreType.DMA,
            pltpu.SemaphoreType.DMA,
            pltpu.SemaphoreType.DMA,
        ],
    )
    def emb_kernel(table_hbm, ids_hbm, out_hbm,
                   ids_v, rows0, rows1, ot0, ot1,
                   gsem0, gsem1, osem0, osem1):
        wid = lax.axis_index("s") * nc + lax.axis_index("c")
        bufs = ((rows0, ot0, gsem0, osem0), (rows1, ot1, gsem1, osem1))

        pltpu.sync_copy(ids_hbm.at[wid], ids_v)

        def gather(b, l):
            rows_v, _, gsem, _ = bufs[b]
            return pltpu.make_async_copy(table_hbm.at[ids_v.at[l]],
                                         rows_v, gsem)

        def out_copies(b, l):
            _, ot, _, osem = bufs[b]
            return [pltpu.make_async_copy(ot.at[pl.ds(eg * 8, 8)],
                                          out_hbm.at[l, eg, wid], osem)
                    for eg in range(ne)]

        iota = lax.iota(jnp.int32, _LANES)
        ridx = [iota + i * _LANES for i in range(_BG // _LANES)]

        for b in range(2):
            gather(b, b).start()

        @pl.loop(0, seq, step=2)
        def _l_loop(g):
            for b in range(2):
                l = g + b
                rows_v, ot, _, _ = bufs[b]
                gather(b, l).wait()

                @pl.when(l >= 2)
                def _drain():
                    for cp in out_copies(b, l):
                        cp.wait()

                @pl.loop(0, _LANES)
                def _transpose(d):
                    perm = lax.rem(iota + d, jnp.int32(_LANES))
                    for e0 in range(0, emb, _LANES):
                        ce = perm + e0
                        for i in range(_BG // _LANES):
                            v = plsc.load_gather(rows_v, [ridx[i], ce])
                            plsc.store_scatter(ot, [ce, ridx[i]], v)

                for cp in out_copies(b, l):
                    cp.start()

                @pl.when(l + 2 < seq)
                def _prefetch():
                    gather(b, l + 2).start()

        for b in range(2):
            for cp in out_copies(b, seq - 2 + b):
                cp.wait()

    return emb_kernel


def kernel(token_ids, table):
    bsz, seq = token_ids.shape
    vocab, emb = table.shape
    nw = bsz // _BG
    table_pad = _build_relayout(vocab, emb)(table.T)
    ids = jnp.transpose(token_ids.T.reshape(seq, nw, _BG), (1, 0, 2))
    ids = ids.astype(jnp.int32)
    out5 = _build_gather(vocab, emb, bsz, seq)(table_pad, ids)
    out = jnp.transpose(out5, (2, 4, 0, 1, 3)).reshape(bsz, seq, emb)
    return out
